# Initial kernel scaffold; baseline (speedup 1.0000x reference)
import jax
import jax.numpy as jnp
from jax.experimental import pallas as pl


def kernel(x, centroids):
    raise NotImplementedError("write your pallas kernel here")


# speedup vs baseline: 1.0801x; 1.0801x over previous
import functools

import jax
import jax.numpy as jnp
from jax import lax
from jax.experimental import pallas as pl
from jax.experimental.pallas import tpu as pltpu
from jax.experimental.pallas import tpu_sc as plsc

N_E = 8192
E_DIM = 32
BETA = 0.25
N_TOK = 8192
ROW_BLK = 1024
COL_CHUNK = 512
WINDOW = 2048
N_ROW_BLKS = N_TOK // ROW_BLK
N_COL_CHUNKS = N_E // COL_CHUNK


def _argmin_body(xt_ref, c_ref, cn_ref, xn_ref, idx_ref, dsum_ref):
    i = pl.program_id(0)
    xt = xt_ref[...]
    xn = xn_ref[...]

    best_val = jnp.full((1, ROW_BLK), jnp.inf, jnp.float32)
    best_true = jnp.zeros((1, ROW_BLK), jnp.float32)
    best_idx = jnp.zeros((1, ROW_BLK), jnp.int32)
    sub_per_win = WINDOW // COL_CHUNK
    for g in range(N_E // WINDOW):
        gv = jnp.full((1, ROW_BLK), jnp.inf, jnp.float32)
        gi = jnp.zeros((1, ROW_BLK), jnp.int32)
        for j4 in range(sub_per_win):
            j = g * sub_per_win + j4
            cj = c_ref[pl.ds(j * COL_CHUNK, COL_CHUNK), :]
            cnj = cn_ref[pl.ds(j * COL_CHUNK, COL_CHUNK), :]
            xy = lax.dot_general(cj, xt, (((1,), (0,)), ((), ())),
                                 preferred_element_type=jnp.float32)
            d = (xn + cnj) - 2.0 * xy
            minv = jnp.min(d, axis=0, keepdims=True)
            ids = lax.broadcasted_iota(jnp.int32, (COL_CHUNK, ROW_BLK), 0)
            loc = jnp.min(jnp.where(d == minv, ids, jnp.int32(2**30)),
                          axis=0, keepdims=True) + j * COL_CHUNK
            upd = minv < gv
            gi = jnp.where(upd, loc, gi)
            gv = jnp.where(upd, minv, gv)
        upd = gv < best_val
        best_idx = jnp.where(upd, gi, best_idx)
        best_true = jnp.where(upd, gv, best_true)
        best_val = jnp.where(
            upd, gv.astype(jnp.bfloat16).astype(jnp.float32), best_val)

    idx_ref[...] = best_idx.reshape(1, 1, ROW_BLK)
    partial = jnp.sum(best_true, axis=1, keepdims=True)

    @pl.when(i == 0)
    def _init():
        dsum_ref[...] = partial

    @pl.when(i > 0)
    def _acc():
        dsum_ref[...] = dsum_ref[...] + partial


def _argmin_call(xt, centroids, cn, xn):
    return pl.pallas_call(
        _argmin_body,
        grid=(N_ROW_BLKS,),
        in_specs=[
            pl.BlockSpec((E_DIM, ROW_BLK), lambda i: (0, i)),
            pl.BlockSpec((N_E, E_DIM), lambda i: (0, 0)),
            pl.BlockSpec((N_E, 1), lambda i: (0, 0)),
            pl.BlockSpec((1, ROW_BLK), lambda i: (0, i)),
        ],
        out_specs=[
            pl.BlockSpec((1, 1, ROW_BLK), lambda i: (i, 0, 0)),
            pl.BlockSpec((1, 1), lambda i: (0, 0)),
        ],
        out_shape=[
            jax.ShapeDtypeStruct((N_ROW_BLKS, 1, ROW_BLK), jnp.int32),
            jax.ShapeDtypeStruct((1, 1), jnp.float32),
        ],
    )(xt, centroids, cn, xn)


PAD_D = 128
GATHER_CHUNK = 128


def _make_sc_gather():
    info = plsc.get_sparse_core_info()
    nw = info.num_cores * info.num_subcores
    b_per_w = N_TOK // nw
    n_chunks = b_per_w // GATHER_CHUNK
    mesh = plsc.VectorSubcoreMesh(core_axis_name="c", subcore_axis_name="s")

    @functools.partial(
        pl.kernel, mesh=mesh,
        out_type=jax.ShapeDtypeStruct((N_TOK, PAD_D), jnp.float32),
        scratch_types=[
            pltpu.VMEM((GATHER_CHUNK,), jnp.int32),
            pltpu.VMEM((GATHER_CHUNK, PAD_D), jnp.float32),
            pltpu.SemaphoreType.DMA,
        ],
    )
    def gather(table_hbm, idx_hbm, out_hbm, idx_v, rows_v, sem):
        wid = lax.axis_index("s") * info.num_cores + lax.axis_index("c")
        base = wid * b_per_w
        for k in range(n_chunks):
            off = base + k * GATHER_CHUNK
            pltpu.sync_copy(idx_hbm.at[pl.ds(off, GATHER_CHUNK)], idx_v)
            pltpu.async_copy(table_hbm.at[idx_v], rows_v, sem).wait()
            pltpu.sync_copy(rows_v, out_hbm.at[pl.ds(off, GATHER_CHUNK)])

    return gather


def kernel(x, centroids):
    latent = x.reshape(-1, E_DIM)
    xt = latent.T.astype(jnp.bfloat16)
    cb = centroids.astype(jnp.bfloat16)
    xn = jnp.sum(latent ** 2, axis=1, keepdims=True).T
    cn = jnp.sum(centroids ** 2, axis=1, keepdims=True)

    idx3, dsum = _argmin_call(xt, cb, cn, xn)
    indices = idx3.reshape(N_ROW_BLKS * ROW_BLK)

    table = jnp.zeros((N_E, PAD_D), jnp.float32).at[:, :E_DIM].set(centroids)
    x_q = _make_sc_gather()(table, indices)[:, :E_DIM].reshape(x.shape)

    loss = dsum[0, 0] * ((1.0 + BETA) / (N_TOK * E_DIM))
    return (x_q, loss, indices.reshape(x.shape[:-1]))


# speedup vs baseline: 1.1546x; 1.0690x over previous
import functools

import jax
import jax.numpy as jnp
from jax import lax
from jax.experimental import pallas as pl
from jax.experimental.pallas import tpu as pltpu
from jax.experimental.pallas import tpu_sc as plsc

N_E = 8192
E_DIM = 32
BETA = 0.25
N_TOK = 8192
ROW_BLK = 1024
COL_CHUNK = 512
WINDOW = 2048
N_ROW_BLKS = N_TOK // ROW_BLK
N_COL_CHUNKS = N_E // COL_CHUNK


def _argmin_body(xt_ref, c_ref, cn_ref, xn_ref, idx_ref, dsum_ref):
    i = pl.program_id(0)
    xt = xt_ref[...]
    xn = xn_ref[...]

    best_val = jnp.full((1, ROW_BLK), jnp.inf, jnp.float32)
    best_true = jnp.zeros((1, ROW_BLK), jnp.float32)
    best_idx = jnp.zeros((1, ROW_BLK), jnp.int32)
    sub_per_win = WINDOW // COL_CHUNK
    for g in range(N_E // WINDOW):
        gv = jnp.full((1, ROW_BLK), jnp.inf, jnp.float32)
        gi = jnp.zeros((1, ROW_BLK), jnp.int32)
        for j4 in range(sub_per_win):
            j = g * sub_per_win + j4
            cj = c_ref[pl.ds(j * COL_CHUNK, COL_CHUNK), :]
            cnj = cn_ref[pl.ds(j * COL_CHUNK, COL_CHUNK), :]
            xy2 = lax.dot_general(cj, xt, (((1,), (0,)), ((), ())),
                                  preferred_element_type=jnp.float32)
            d = (xn + cnj) - xy2
            minv = jnp.min(d, axis=0, keepdims=True)
            ids = lax.broadcasted_iota(jnp.int32, (COL_CHUNK, ROW_BLK), 0)
            loc = jnp.min(jnp.where(d == minv, ids, jnp.int32(2**30)),
                          axis=0, keepdims=True) + j * COL_CHUNK
            upd = minv < gv
            gi = jnp.where(upd, loc, gi)
            gv = jnp.where(upd, minv, gv)
        upd = gv < best_val
        best_idx = jnp.where(upd, gi, best_idx)
        best_true = jnp.where(upd, gv, best_true)
        best_val = jnp.where(
            upd, gv.astype(jnp.bfloat16).astype(jnp.float32), best_val)

    idx_ref[...] = best_idx.reshape(1, 1, ROW_BLK)
    partial = jnp.sum(best_true, axis=1, keepdims=True)

    @pl.when(i == 0)
    def _init():
        dsum_ref[...] = partial

    @pl.when(i > 0)
    def _acc():
        dsum_ref[...] = dsum_ref[...] + partial


def _argmin_call(xt, centroids, cn, xn):
    return pl.pallas_call(
        _argmin_body,
        grid=(N_ROW_BLKS,),
        in_specs=[
            pl.BlockSpec((E_DIM, ROW_BLK), lambda i: (0, i)),
            pl.BlockSpec((N_E, E_DIM), lambda i: (0, 0)),
            pl.BlockSpec((N_E, 1), lambda i: (0, 0)),
            pl.BlockSpec((1, ROW_BLK), lambda i: (0, i)),
        ],
        out_specs=[
            pl.BlockSpec((1, 1, ROW_BLK), lambda i: (i, 0, 0)),
            pl.BlockSpec((1, 1), lambda i: (0, 0)),
        ],
        out_shape=[
            jax.ShapeDtypeStruct((N_ROW_BLKS, 1, ROW_BLK), jnp.int32),
            jax.ShapeDtypeStruct((1, 1), jnp.float32),
        ],
    )(xt, centroids, cn, xn)


PAD_D = 128
GATHER_CHUNK = 128


def _make_sc_gather():
    info = plsc.get_sparse_core_info()
    nw = info.num_cores * info.num_subcores
    b_per_w = N_TOK // nw
    n_chunks = b_per_w // GATHER_CHUNK
    mesh = plsc.VectorSubcoreMesh(core_axis_name="c", subcore_axis_name="s")

    @functools.partial(
        pl.kernel, mesh=mesh,
        out_type=jax.ShapeDtypeStruct((N_TOK, PAD_D), jnp.float32),
        scratch_types=[
            pltpu.VMEM((GATHER_CHUNK,), jnp.int32),
            pltpu.VMEM((GATHER_CHUNK, PAD_D), jnp.float32),
            pltpu.SemaphoreType.DMA,
        ],
    )
    def gather(table_hbm, idx_hbm, out_hbm, idx_v, rows_v, sem):
        wid = lax.axis_index("s") * info.num_cores + lax.axis_index("c")
        base = wid * b_per_w
        for k in range(n_chunks):
            off = base + k * GATHER_CHUNK
            pltpu.sync_copy(idx_hbm.at[pl.ds(off, GATHER_CHUNK)], idx_v)
            pltpu.async_copy(table_hbm.at[idx_v], rows_v, sem).wait()
            pltpu.sync_copy(rows_v, out_hbm.at[pl.ds(off, GATHER_CHUNK)])

    return gather


def kernel(x, centroids):
    latent = x.reshape(-1, E_DIM)
    xt = latent.T.astype(jnp.bfloat16)
    cb = (2.0 * centroids).astype(jnp.bfloat16)
    xn = jnp.sum(latent ** 2, axis=1, keepdims=True).T
    cn = jnp.sum(centroids ** 2, axis=1, keepdims=True)

    idx3, dsum = _argmin_call(xt, cb, cn, xn)
    indices = idx3.reshape(N_ROW_BLKS * ROW_BLK)

    table = jnp.zeros((N_E, PAD_D), jnp.float32).at[:, :E_DIM].set(centroids)
    x_q = _make_sc_gather()(table, indices)[:, :E_DIM].reshape(x.shape)

    loss = dsum[0, 0] * ((1.0 + BETA) / (N_TOK * E_DIM))
    return (x_q, loss, indices.reshape(x.shape[:-1]))


# speedup vs baseline: 1.5324x; 1.3271x over previous
import functools

import jax
import jax.numpy as jnp
from jax import lax
from jax.experimental import pallas as pl
from jax.experimental.pallas import tpu as pltpu
from jax.experimental.pallas import tpu_sc as plsc

N_E = 8192
E_DIM = 32
BETA = 0.25
N_TOK = 8192
ROW_BLK = 1024
COL_CHUNK = 512
WINDOW = 2048
N_ROW_BLKS = N_TOK // ROW_BLK
N_COL_CHUNKS = N_E // COL_CHUNK


def _argmin_body(xt_ref, c_ref, cn_ref, xn_ref, idx_ref, dsum_ref):
    i = pl.program_id(0)
    xt = xt_ref[...]
    xn = xn_ref[...]

    best_val = jnp.full((1, ROW_BLK), jnp.inf, jnp.float32)
    best_true = jnp.zeros((1, ROW_BLK), jnp.float32)
    best_idx = jnp.zeros((1, ROW_BLK), jnp.int32)
    sub_per_win = WINDOW // COL_CHUNK
    strips = COL_CHUNK // 8
    rsub = lax.broadcasted_iota(jnp.int32, (8, ROW_BLK), 0)
    for g in range(N_E // WINDOW):
        runmin = jnp.full((8, ROW_BLK), jnp.inf, jnp.float32)
        runstrip = jnp.zeros((8, ROW_BLK), jnp.int32)
        for j4 in range(sub_per_win):
            j = g * sub_per_win + j4
            cj = c_ref[pl.ds(j * COL_CHUNK, COL_CHUNK), :]
            cnj = cn_ref[pl.ds(j * COL_CHUNK, COL_CHUNK), :]
            xy2 = lax.dot_general(cj, xt, (((1,), (0,)), ((), ())),
                                  preferred_element_type=jnp.float32)
            for s in range(strips):
                d_s = (xn + cnj[s * 8:(s + 1) * 8, :]) - xy2[s * 8:(s + 1) * 8, :]
                cmp = d_s < runmin
                runmin = jnp.where(cmp, d_s, runmin)
                sid = jnp.full((8, ROW_BLK), j4 * strips + s, jnp.int32)
                runstrip = jnp.where(cmp, sid, runstrip)
        gv = jnp.min(runmin, axis=0, keepdims=True)
        cand = runstrip * 8 + rsub
        gi = jnp.min(jnp.where(runmin == gv, cand, jnp.int32(2**30)),
                     axis=0, keepdims=True) + g * WINDOW
        upd = gv < best_val
        best_idx = jnp.where(upd, gi, best_idx)
        best_true = jnp.where(upd, gv, best_true)
        best_val = jnp.where(
            upd, gv.astype(jnp.bfloat16).astype(jnp.float32), best_val)

    idx_ref[...] = best_idx.reshape(1, 1, ROW_BLK)
    partial = jnp.sum(best_true, axis=1, keepdims=True)

    @pl.when(i == 0)
    def _init():
        dsum_ref[...] = partial

    @pl.when(i > 0)
    def _acc():
        dsum_ref[...] = dsum_ref[...] + partial


def _argmin_call(xt, centroids, cn, xn):
    return pl.pallas_call(
        _argmin_body,
        grid=(N_ROW_BLKS,),
        in_specs=[
            pl.BlockSpec((E_DIM, ROW_BLK), lambda i: (0, i)),
            pl.BlockSpec((N_E, E_DIM), lambda i: (0, 0)),
            pl.BlockSpec((N_E, 1), lambda i: (0, 0)),
            pl.BlockSpec((1, ROW_BLK), lambda i: (0, i)),
        ],
        out_specs=[
            pl.BlockSpec((1, 1, ROW_BLK), lambda i: (i, 0, 0)),
            pl.BlockSpec((1, 1), lambda i: (0, 0)),
        ],
        out_shape=[
            jax.ShapeDtypeStruct((N_ROW_BLKS, 1, ROW_BLK), jnp.int32),
            jax.ShapeDtypeStruct((1, 1), jnp.float32),
        ],
    )(xt, centroids, cn, xn)


PAD_D = 128
GATHER_CHUNK = 128


def _make_sc_gather():
    info = plsc.get_sparse_core_info()
    nw = info.num_cores * info.num_subcores
    b_per_w = N_TOK // nw
    n_chunks = b_per_w // GATHER_CHUNK
    mesh = plsc.VectorSubcoreMesh(core_axis_name="c", subcore_axis_name="s")

    @functools.partial(
        pl.kernel, mesh=mesh,
        out_type=jax.ShapeDtypeStruct((N_TOK, PAD_D), jnp.float32),
        scratch_types=[
            pltpu.VMEM((GATHER_CHUNK,), jnp.int32),
            pltpu.VMEM((GATHER_CHUNK, PAD_D), jnp.float32),
            pltpu.SemaphoreType.DMA,
        ],
    )
    def gather(table_hbm, idx_hbm, out_hbm, idx_v, rows_v, sem):
        wid = lax.axis_index("s") * info.num_cores + lax.axis_index("c")
        base = wid * b_per_w
        for k in range(n_chunks):
            off = base + k * GATHER_CHUNK
            pltpu.sync_copy(idx_hbm.at[pl.ds(off, GATHER_CHUNK)], idx_v)
            pltpu.async_copy(table_hbm.at[idx_v], rows_v, sem).wait()
            pltpu.sync_copy(rows_v, out_hbm.at[pl.ds(off, GATHER_CHUNK)])

    return gather


def kernel(x, centroids):
    latent = x.reshape(-1, E_DIM)
    xt = latent.T.astype(jnp.bfloat16)
    cb = (2.0 * centroids).astype(jnp.bfloat16)
    xn = jnp.sum(latent ** 2, axis=1, keepdims=True).T
    cn = jnp.sum(centroids ** 2, axis=1, keepdims=True)

    idx3, dsum = _argmin_call(xt, cb, cn, xn)
    indices = idx3.reshape(N_ROW_BLKS * ROW_BLK)

    table = jnp.zeros((N_E, PAD_D), jnp.float32).at[:, :E_DIM].set(centroids)
    x_q = _make_sc_gather()(table, indices)[:, :E_DIM].reshape(x.shape)

    loss = dsum[0, 0] * ((1.0 + BETA) / (N_TOK * E_DIM))
    return (x_q, loss, indices.reshape(x.shape[:-1]))
